# baseline (speedup 1.0000x reference)
import functools

import jax
import jax.numpy as jnp
from jax import lax
from jax.experimental import pallas as pl
from jax.experimental.pallas import tpu as pltpu
from jax.experimental.pallas import tpu_sc as plsc

NUM_CORES = 2
NUM_SUBCORES = 16
LANES = 16
NW = NUM_CORES * NUM_SUBCORES

BATCH = 16384
DIM = 128
ROWS_PER_W = BATCH // NW
CHUNK = 128
NCHUNK = ROWS_PER_W // CHUNK
GROUPS = CHUNK // LANES


def _body(uids_hbm, iids_hbm, utab_hbm, itab_hbm, out_hbm,
          uidx_v, iidx_v, ub0, ub1, ib0, ib1, out_v,
          su0, su1, si0, si1):
  wid = lax.axis_index("c") * NUM_SUBCORES + lax.axis_index("s")

  pltpu.sync_copy(uids_hbm.at[wid], uidx_v)
  pltpu.sync_copy(iids_hbm.at[wid], iidx_v)

  ubufs = (ub0, ub1)
  ibufs = (ib0, ib1)
  usems = (su0, su1)
  isems = (si0, si1)

  def gather(c):
    b = c % 2
    ud = pltpu.make_async_copy(utab_hbm.at[uidx_v.at[c]], ubufs[b], usems[b])
    idd = pltpu.make_async_copy(itab_hbm.at[iidx_v.at[c]], ibufs[b], isems[b])
    ud.start()
    idd.start()
    return ud, idd

  def wait(c):
    b = c % 2
    pltpu.make_async_copy(utab_hbm.at[uidx_v.at[c]], ubufs[b], usems[b]).wait()
    pltpu.make_async_copy(itab_hbm.at[iidx_v.at[c]], ibufs[b], isems[b]).wait()

  gather(0)
  for c in range(NCHUNK):
    if c + 1 < NCHUNK:
      gather(c + 1)
    wait(c)
    ub = ubufs[c % 2]
    ib = ibufs[c % 2]

    def group(g, _, ub=ub, ib=ib, c=c):
      rows = g * LANES + lax.iota(jnp.int32, LANES)
      accs = [jnp.zeros((LANES,), jnp.float32) for _ in range(4)]
      for d in range(DIM):
        cols = jnp.full((LANES,), d, jnp.int32)
        u16 = plsc.load_gather(ub, [rows, cols])
        i16 = plsc.load_gather(ib, [rows, cols])
        accs[d % 4] = accs[d % 4] + u16 * i16
      out16 = (accs[0] + accs[1]) + (accs[2] + accs[3])
      off = pl.multiple_of(c * CHUNK + g * LANES, LANES)
      out_v[pl.ds(off, LANES)] = out16
      return 0

    lax.fori_loop(0, GROUPS, group, 0)

  pltpu.sync_copy(out_v, out_hbm.at[wid])


@functools.partial(jax.jit, static_argnums=())
def kernel(user_ids, item_ids, user_table, item_table):
  uids = user_ids.astype(jnp.int32).reshape(NW, NCHUNK, CHUNK)
  iids = item_ids.astype(jnp.int32).reshape(NW, NCHUNK, CHUNK)

  mesh = plsc.VectorSubcoreMesh(core_axis_name="c", subcore_axis_name="s")
  k = pl.kernel(
      _body,
      out_type=jax.ShapeDtypeStruct((NW, ROWS_PER_W), jnp.float32),
      mesh=mesh,
      compiler_params=pltpu.CompilerParams(needs_layout_passes=False),
      scratch_types=[
          pltpu.VMEM((NCHUNK, CHUNK), jnp.int32),
          pltpu.VMEM((NCHUNK, CHUNK), jnp.int32),
          pltpu.VMEM((CHUNK, DIM), jnp.float32),
          pltpu.VMEM((CHUNK, DIM), jnp.float32),
          pltpu.VMEM((CHUNK, DIM), jnp.float32),
          pltpu.VMEM((CHUNK, DIM), jnp.float32),
          pltpu.VMEM((ROWS_PER_W,), jnp.float32),
          pltpu.SemaphoreType.DMA,
          pltpu.SemaphoreType.DMA,
          pltpu.SemaphoreType.DMA,
          pltpu.SemaphoreType.DMA,
      ],
  )
  out = k(uids, iids, user_table, item_table)
  return out.reshape(BATCH)


# speedup vs baseline: 1.3484x; 1.3484x over previous
import functools

import jax
import jax.numpy as jnp
from jax import lax
from jax.experimental import pallas as pl
from jax.experimental.pallas import tpu as pltpu
from jax.experimental.pallas import tpu_sc as plsc

NUM_CORES = 2
NUM_SUBCORES = 16
LANES = 16
NW = NUM_CORES * NUM_SUBCORES

BATCH = 16384
DIM = 128
ROWS_PER_W = BATCH // NW
CHUNK = 128
NCHUNK = ROWS_PER_W // CHUNK
GROUPS = CHUNK // LANES


def _body(uids_hbm, iids_hbm, utab_hbm, itab_hbm, out_hbm,
          uidx_v, iidx_v, ub0, ub1, ib0, ib1, out_v,
          su0, su1, si0, si1):
  wid = lax.axis_index("c") * NUM_SUBCORES + lax.axis_index("s")

  pltpu.sync_copy(uids_hbm.at[wid], uidx_v)
  pltpu.sync_copy(iids_hbm.at[wid], iidx_v)

  ubufs = (ub0, ub1)
  ibufs = (ib0, ib1)
  usems = (su0, su1)
  isems = (si0, si1)

  def gather(c):
    b = c % 2
    ud = pltpu.make_async_copy(utab_hbm.at[uidx_v.at[c]], ubufs[b], usems[b])
    idd = pltpu.make_async_copy(itab_hbm.at[iidx_v.at[c]], ibufs[b], isems[b])
    ud.start()
    idd.start()
    return ud, idd

  def wait(c):
    b = c % 2
    pltpu.make_async_copy(utab_hbm.at[uidx_v.at[c]], ubufs[b], usems[b]).wait()
    pltpu.make_async_copy(itab_hbm.at[iidx_v.at[c]], ibufs[b], isems[b]).wait()

  gather(0)
  for c in range(NCHUNK):
    if c + 1 < NCHUNK:
      gather(c + 1)
    wait(c)
    ub = ubufs[c % 2]
    ib = ibufs[c % 2]

    def group(g, _, ub=ub, ib=ib, c=c):
      rows = g * LANES + lax.iota(jnp.int32, LANES)
      zero = jnp.zeros((LANES,), jnp.float32)

      def dstep(t, accs, ub=ub, ib=ib, rows=rows):
        a = list(accs)
        base = t * 8
        for q in range(8):
          cols = jnp.full((LANES,), base + q, jnp.int32)
          u16 = plsc.load_gather(ub, [rows, cols])
          i16 = plsc.load_gather(ib, [rows, cols])
          a[q % 4] = a[q % 4] + u16 * i16
        return tuple(a)

      accs = lax.fori_loop(0, DIM // 8, dstep, (zero, zero, zero, zero))
      out16 = (accs[0] + accs[1]) + (accs[2] + accs[3])
      off = pl.multiple_of(c * CHUNK + g * LANES, LANES)
      out_v[pl.ds(off, LANES)] = out16
      return 0

    lax.fori_loop(0, GROUPS, group, 0)

  pltpu.sync_copy(out_v, out_hbm.at[wid])


@functools.partial(jax.jit, static_argnums=())
def kernel(user_ids, item_ids, user_table, item_table):
  uids = user_ids.astype(jnp.int32).reshape(NW, NCHUNK, CHUNK)
  iids = item_ids.astype(jnp.int32).reshape(NW, NCHUNK, CHUNK)

  mesh = plsc.VectorSubcoreMesh(core_axis_name="c", subcore_axis_name="s")
  k = pl.kernel(
      _body,
      out_type=jax.ShapeDtypeStruct((NW, ROWS_PER_W), jnp.float32),
      mesh=mesh,
      compiler_params=pltpu.CompilerParams(needs_layout_passes=False),
      scratch_types=[
          pltpu.VMEM((NCHUNK, CHUNK), jnp.int32),
          pltpu.VMEM((NCHUNK, CHUNK), jnp.int32),
          pltpu.VMEM((CHUNK, DIM), jnp.float32),
          pltpu.VMEM((CHUNK, DIM), jnp.float32),
          pltpu.VMEM((CHUNK, DIM), jnp.float32),
          pltpu.VMEM((CHUNK, DIM), jnp.float32),
          pltpu.VMEM((ROWS_PER_W,), jnp.float32),
          pltpu.SemaphoreType.DMA,
          pltpu.SemaphoreType.DMA,
          pltpu.SemaphoreType.DMA,
          pltpu.SemaphoreType.DMA,
      ],
  )
  out = k(uids, iids, user_table, item_table)
  return out.reshape(BATCH)


# speedup vs baseline: 1.3505x; 1.0016x over previous
import functools

import jax
import jax.numpy as jnp
from jax import lax
from jax.experimental import pallas as pl
from jax.experimental.pallas import tpu as pltpu
from jax.experimental.pallas import tpu_sc as plsc

NUM_CORES = 2
NUM_SUBCORES = 16
LANES = 16
NW = NUM_CORES * NUM_SUBCORES

BATCH = 16384
DIM = 128
ROWS_PER_W = BATCH // NW
CHUNK = 64
NCHUNK = ROWS_PER_W // CHUNK
GROUPS = CHUNK // LANES


def _body(uids_hbm, iids_hbm, utab_hbm, itab_hbm, out_hbm,
          uidx_v, iidx_v, ub0, ub1, ib0, ib1, out_v,
          su0, su1, si0, si1):
  wid = lax.axis_index("c") * NUM_SUBCORES + lax.axis_index("s")

  pltpu.sync_copy(uids_hbm.at[wid], uidx_v)
  pltpu.sync_copy(iids_hbm.at[wid], iidx_v)

  ubufs = (ub0, ub1)
  ibufs = (ib0, ib1)
  usems = (su0, su1)
  isems = (si0, si1)

  def gather(c):
    b = c % 2
    pltpu.make_async_copy(
        utab_hbm.at[uidx_v.at[c]], ubufs[b], usems[b]).start()
    pltpu.make_async_copy(
        itab_hbm.at[iidx_v.at[c]], ibufs[b], isems[b]).start()

  def wait(c):
    b = c % 2
    pltpu.make_async_copy(utab_hbm.at[uidx_v.at[c]], ubufs[b], usems[b]).wait()
    pltpu.make_async_copy(itab_hbm.at[iidx_v.at[c]], ibufs[b], isems[b]).wait()

  gather(0)
  for c in range(NCHUNK):
    if c + 1 < NCHUNK:
      gather(c + 1)
    wait(c)
    ub = ubufs[c % 2]
    ib = ibufs[c % 2]

    def group(g, _, ub=ub, ib=ib, c=c):
      rows = g * LANES + lax.iota(jnp.int32, LANES)
      zero = jnp.zeros((LANES,), jnp.float32)

      def dstep(t, accs, ub=ub, ib=ib, rows=rows):
        a = list(accs)
        base = t * 8
        for q in range(8):
          cols = jnp.full((LANES,), base + q, jnp.int32)
          u16 = plsc.load_gather(ub, [rows, cols])
          i16 = plsc.load_gather(ib, [rows, cols])
          a[q % 4] = a[q % 4] + u16 * i16
        return tuple(a)

      accs = lax.fori_loop(0, DIM // 8, dstep, (zero, zero, zero, zero))
      out16 = (accs[0] + accs[1]) + (accs[2] + accs[3])
      off = pl.multiple_of(c * CHUNK + g * LANES, LANES)
      out_v[pl.ds(off, LANES)] = out16
      return 0

    lax.fori_loop(0, GROUPS, group, 0)

  pltpu.sync_copy(out_v, out_hbm.at[wid])


@functools.partial(jax.jit, static_argnums=())
def kernel(user_ids, item_ids, user_table, item_table):
  uids = user_ids.astype(jnp.int32).reshape(NW, NCHUNK, CHUNK)
  iids = item_ids.astype(jnp.int32).reshape(NW, NCHUNK, CHUNK)

  mesh = plsc.VectorSubcoreMesh(core_axis_name="c", subcore_axis_name="s")
  k = pl.kernel(
      _body,
      out_type=jax.ShapeDtypeStruct((NW, ROWS_PER_W), jnp.float32),
      mesh=mesh,
      compiler_params=pltpu.CompilerParams(needs_layout_passes=False),
      scratch_types=[
          pltpu.VMEM((NCHUNK, CHUNK), jnp.int32),
          pltpu.VMEM((NCHUNK, CHUNK), jnp.int32),
          pltpu.VMEM((CHUNK, DIM), jnp.float32),
          pltpu.VMEM((CHUNK, DIM), jnp.float32),
          pltpu.VMEM((CHUNK, DIM), jnp.float32),
          pltpu.VMEM((CHUNK, DIM), jnp.float32),
          pltpu.VMEM((ROWS_PER_W,), jnp.float32),
          pltpu.SemaphoreType.DMA,
          pltpu.SemaphoreType.DMA,
          pltpu.SemaphoreType.DMA,
          pltpu.SemaphoreType.DMA,
      ],
  )
  out = k(uids, iids, user_table, item_table)
  return out.reshape(BATCH)
